# Initial kernel scaffold; baseline (speedup 1.0000x reference)
import jax
import jax.numpy as jnp
from jax.experimental import pallas as pl


def kernel(coord_out, coord_gt, valid):
    raise NotImplementedError("write your pallas kernel here")


# speedup vs baseline: 1.2145x; 1.2145x over previous
import dataclasses
import functools

import jax
import jax.numpy as jnp
from jax import lax
from jax.experimental import pallas as pl
from jax.experimental.pallas import tpu as pltpu
from jax.experimental.pallas import tpu_sc as plsc

B = 8192
F = 128
L = 16
GROUPS = F // L
CW = 9 * F
VW = 3 * F
CB = 8
EPS2 = 1e-24


def _rsqrt(s):
    i = lax.bitcast_convert_type(s, jnp.int32)
    i = jnp.int32(0x5F3759DF) - lax.shift_right_logical(i, 1)
    y = lax.bitcast_convert_type(i, jnp.float32)
    sh = 0.5 * s
    for _ in range(3):
        y = y * (1.5 - sh * y * y)
    return y


def _nvl_block(co_v, cg_v, va_v, out_v):
    lane = lax.iota(jnp.int32, L)
    col9 = lane * 9
    col3 = lane * 3
    zero16 = jnp.zeros((L,), jnp.int32)

    @pl.loop(0, CB)
    def _row(b):
        row = zero16 + b

        @pl.loop(0, GROUPS)
        def _group(g):
            base9 = col9 + g * (9 * L)
            base3 = col3 + g * (3 * L)

            def ld(ref, base, off):
                return plsc.load_gather(ref, [row, base + off])

            ox0, oy0, oz0 = (ld(co_v, base9, k) for k in (0, 1, 2))
            ox1, oy1, oz1 = (ld(co_v, base9, k) for k in (3, 4, 5))
            ox2, oy2, oz2 = (ld(co_v, base9, k) for k in (6, 7, 8))
            a1x, a1y, a1z = ox1 - ox0, oy1 - oy0, oz1 - oz0
            a2x, a2y, a2z = ox2 - ox0, oy2 - oy0, oz2 - oz0
            a3x, a3y, a3z = a2x - a1x, a2y - a1y, a2z - a1z
            r1 = _rsqrt(jnp.maximum(a1x * a1x + a1y * a1y + a1z * a1z, EPS2))
            r2 = _rsqrt(jnp.maximum(a2x * a2x + a2y * a2y + a2z * a2z, EPS2))
            r3 = _rsqrt(jnp.maximum(a3x * a3x + a3y * a3y + a3z * a3z, EPS2))

            gx0, gy0, gz0 = (ld(cg_v, base9, k) for k in (0, 1, 2))
            gx1, gy1, gz1 = (ld(cg_v, base9, k) for k in (3, 4, 5))
            gx2, gy2, gz2 = (ld(cg_v, base9, k) for k in (6, 7, 8))
            e1x, e1y, e1z = gx1 - gx0, gy1 - gy0, gz1 - gz0
            e2x, e2y, e2z = gx2 - gx0, gy2 - gy0, gz2 - gz0
            re1 = _rsqrt(jnp.maximum(e1x * e1x + e1y * e1y + e1z * e1z, EPS2))
            re2 = _rsqrt(jnp.maximum(e2x * e2x + e2y * e2y + e2z * e2z, EPS2))
            q = re1 * re2
            cx = e1y * e2z - e1z * e2y
            cy = e1z * e2x - e1x * e2z
            cz = e1x * e2y - e1y * e2x
            sc = (cx * cx + cy * cy + cz * cz) * q * q
            t = q * _rsqrt(jnp.maximum(sc, EPS2))
            nx, ny, nz = cx * t, cy * t, cz * t

            m = (ld(va_v, base3, 0) * ld(va_v, base3, 1) * ld(va_v, base3, 2))
            m1, m2, m3 = m * r1, m * r2, m * r3
            cos1 = jnp.abs(a1x * nx + a1y * ny + a1z * nz) * m1
            cos2 = jnp.abs(a2x * nx + a2y * ny + a2z * nz) * m2
            cos3 = jnp.abs(a3x * nx + a3y * ny + a3z * nz) * m3

            out_v[b, pl.ds(g * L, L)] = cos1
            out_v[b, pl.ds(F + g * L, L)] = cos2
            out_v[b, pl.ds(2 * F + g * L, L)] = cos3


@jax.jit
def _nvl(co, cg, va):
    mesh = plsc.VectorSubcoreMesh(core_axis_name="core",
                                  subcore_axis_name="subcore")
    cp = pltpu.CompilerParams()
    if "needs_layout_passes" in pltpu.CompilerParams.__dataclass_fields__:
        cp = dataclasses.replace(cp, needs_layout_passes=False)

    @functools.partial(
        pl.kernel,
        out_type=jax.ShapeDtypeStruct((B, VW), jnp.float32),
        mesh=mesh,
        compiler_params=cp,
    )
    def knl(co_hbm, cg_hbm, va_hbm, out_hbm):
        pltpu.emit_pipeline(
            _nvl_block,
            grid=(B // CB,),
            in_specs=[
                pl.BlockSpec((CB, CW), lambda i: (i, 0)),
                pl.BlockSpec((CB, CW), lambda i: (i, 0)),
                pl.BlockSpec((CB, VW), lambda i: (i, 0)),
            ],
            out_specs=[pl.BlockSpec((CB, VW), lambda i: (i, 0))],
            core_axis_name=("core", "subcore"),
            dimension_semantics=(pltpu.PARALLEL,),
        )(co_hbm, cg_hbm, va_hbm, out_hbm)

    return knl(co, cg, va)


def kernel(coord_out, coord_gt, valid):
    co = coord_out.reshape(B, CW)
    cg = coord_gt.reshape(B, CW)
    va = valid.reshape(B, VW)
    return _nvl(co, cg, va).reshape(B, VW, 1)
